# Initial kernel scaffold; baseline (speedup 1.0000x reference)
import functools

import jax
import jax.numpy as jnp
from jax import lax
from jax.experimental import pallas as pl
from jax.experimental.pallas import tpu as pltpu
from jax.experimental.pallas import tpu_sc as plsc

N = 10000
E = 160000
D = 256
H = 64
C = 10

NC = 2
NS = 16
N_PAD = 10240
ROWS_PER_TILE = N_PAD // NS
CH = 128
EPT = 5120
NCHUNK = EPT // CH
EPC = EPT * NS
E_PAD = EPC * NC
DEG_W = 16

RB = 400
GRID = N // RB

_mesh = plsc.VectorSubcoreMesh(core_axis_name="c", subcore_axis_name="s")


def _zero_fill(ref, ncols):
    def body(r, _):
        for j in range(ncols // 16):
            ref[r, pl.ds(j * 16, 16)] = jnp.zeros((16,), jnp.float32)
        return 0
    lax.fori_loop(0, CH, body, 0)


@functools.partial(
    pl.kernel,
    out_type=jax.ShapeDtypeStruct((NC, N_PAD, DEG_W), jnp.float32),
    mesh=_mesh,
    scratch_types=[
        pltpu.VMEM((CH,), jnp.int32),
        pltpu.VMEM((CH, DEG_W), jnp.float32),
        pltpu.VMEM_SHARED((N_PAD, DEG_W), jnp.float32),
    ],
)
def _sc_degree(dst_hbm, out_hbm, didx, ones_v, acc):
    c = lax.axis_index("c")
    s = lax.axis_index("s")
    _zero_fill(ones_v, DEG_W)
    for k in range(ROWS_PER_TILE // CH):
        pltpu.sync_copy(ones_v, acc.at[pl.ds(s * ROWS_PER_TILE + k * CH, CH)])
    def fill1(r, _):
        ones_v[r, pl.ds(0, 16)] = jnp.full((16,), 1.0, jnp.float32)
        return 0
    lax.fori_loop(0, CH, fill1, 0)
    plsc.subcore_barrier()

    def chunk(i, _):
        base = pl.multiple_of(c * EPC + s * EPT + i * CH, CH)
        pltpu.sync_copy(dst_hbm.at[pl.ds(base, CH)], didx)
        pltpu.sync_copy(ones_v, acc.at[didx], add=True)
        return 0
    lax.fori_loop(0, NCHUNK, chunk, 0)
    plsc.subcore_barrier()
    pltpu.sync_copy(
        acc.at[pl.ds(s * ROWS_PER_TILE, ROWS_PER_TILE)],
        out_hbm.at[c, pl.ds(s * ROWS_PER_TILE, ROWS_PER_TILE)],
    )


@functools.partial(
    pl.kernel,
    out_type=jax.ShapeDtypeStruct((NC, N_PAD, H), jnp.float32),
    mesh=_mesh,
    scratch_types=[
        pltpu.VMEM((CH,), jnp.int32),
        pltpu.VMEM((CH,), jnp.int32),
        pltpu.VMEM((CH, H), jnp.float32),
        pltpu.VMEM_SHARED((N_PAD, H), jnp.float32),
        pltpu.SemaphoreType.DMA,
    ],
)
def _sc_aggregate(g_hbm, src_hbm, dst_hbm, out_hbm, sidx, didx, rows, acc, sem):
    c = lax.axis_index("c")
    s = lax.axis_index("s")
    _zero_fill(rows, H)
    for k in range(ROWS_PER_TILE // CH):
        pltpu.sync_copy(rows, acc.at[pl.ds(s * ROWS_PER_TILE + k * CH, CH)])
    plsc.subcore_barrier()

    def chunk(i, _):
        base = pl.multiple_of(c * EPC + s * EPT + i * CH, CH)
        pltpu.sync_copy(src_hbm.at[pl.ds(base, CH)], sidx)
        pltpu.sync_copy(dst_hbm.at[pl.ds(base, CH)], didx)
        pltpu.async_copy(g_hbm.at[sidx], rows, sem).wait()
        pltpu.sync_copy(rows, acc.at[didx], add=True)
        return 0
    lax.fori_loop(0, NCHUNK, chunk, 0)
    plsc.subcore_barrier()
    pltpu.sync_copy(
        acc.at[pl.ds(s * ROWS_PER_TILE, ROWS_PER_TILE)],
        out_hbm.at[c, pl.ds(s * ROWS_PER_TILE, ROWS_PER_TILE)],
    )


def _dinv_block(deg_ref):
    d = deg_ref[0, :, 0:1] + deg_ref[1, :, 0:1] + 1.0
    return lax.rsqrt(d)


def _tc_mm1_body(x_ref, w_ref, deg_ref, h_ref, g_ref):
    h = jnp.dot(x_ref[...], w_ref[...], preferred_element_type=jnp.float32)
    dinv = _dinv_block(deg_ref)
    h_ref[...] = h
    g_ref[...] = h * dinv


def _tc_layer2_body(h1_ref, agg_ref, deg_ref, w2_ref, b1_ref, h2_ref, g2_ref):
    dinv = _dinv_block(deg_ref)
    agg = agg_ref[0] + agg_ref[1]
    out1 = jnp.maximum(dinv * agg + dinv * dinv * h1_ref[...] + b1_ref[...], 0.0)
    h2 = jnp.dot(out1, w2_ref[...], preferred_element_type=jnp.float32)
    h2_ref[...] = h2
    g2_ref[...] = h2 * dinv


def _tc_final_body(h2_ref, agg_ref, deg_ref, b2_ref, wfc_ref, bfc_ref,
                   out_ref, acc_ref):
    i = pl.program_id(0)
    dinv = _dinv_block(deg_ref)
    agg = agg_ref[0] + agg_ref[1]
    out2 = jnp.maximum(dinv * agg + dinv * dinv * h2_ref[...] + b2_ref[...], 0.0)
    part = jnp.sum(out2, axis=0, keepdims=True)

    @pl.when(i == 0)
    def _():
        acc_ref[...] = part

    @pl.when(i > 0)
    def _():
        acc_ref[...] = acc_ref[...] + part

    @pl.when(i == GRID - 1)
    def _():
        pooled = acc_ref[...] * (1.0 / N)
        out_ref[...] = (
            jnp.dot(pooled, wfc_ref[...], preferred_element_type=jnp.float32)
            + bfc_ref[...]
        )


def kernel(x, edge_index, W1, b1, W2, b2, Wfc, bfc):
    src = edge_index[0]
    dst = edge_index[1]
    pad = E_PAD - E
    src_p = jnp.concatenate([src, jnp.zeros((pad,), jnp.int32)])
    dst_p = jnp.concatenate([dst, jnp.full((pad,), N, jnp.int32)])
    b1r = b1.reshape(1, H)
    b2r = b2.reshape(1, H)
    bfcr = bfc.reshape(1, C)

    deg = _sc_degree(dst_p)

    deg_spec = pl.BlockSpec((NC, RB, DEG_W), lambda i: (0, i, 0))
    agg_spec = pl.BlockSpec((NC, RB, H), lambda i: (0, i, 0))
    row_spec = pl.BlockSpec((RB, H), lambda i: (i, 0))

    h1, g1 = pl.pallas_call(
        _tc_mm1_body,
        grid=(GRID,),
        in_specs=[
            pl.BlockSpec((RB, D), lambda i: (i, 0)),
            pl.BlockSpec((D, H), lambda i: (0, 0)),
            deg_spec,
        ],
        out_specs=[row_spec, row_spec],
        out_shape=[
            jax.ShapeDtypeStruct((N, H), jnp.float32),
            jax.ShapeDtypeStruct((N, H), jnp.float32),
        ],
    )(x, W1, deg)

    agg1 = _sc_aggregate(g1, src_p, dst_p)

    h2, g2 = pl.pallas_call(
        _tc_layer2_body,
        grid=(GRID,),
        in_specs=[
            row_spec,
            agg_spec,
            deg_spec,
            pl.BlockSpec((H, H), lambda i: (0, 0)),
            pl.BlockSpec((1, H), lambda i: (0, 0)),
        ],
        out_specs=[row_spec, row_spec],
        out_shape=[
            jax.ShapeDtypeStruct((N, H), jnp.float32),
            jax.ShapeDtypeStruct((N, H), jnp.float32),
        ],
    )(h1, agg1, deg, W2, b1r)

    agg2 = _sc_aggregate(g2, src_p, dst_p)

    out = pl.pallas_call(
        _tc_final_body,
        grid=(GRID,),
        in_specs=[
            row_spec,
            agg_spec,
            deg_spec,
            pl.BlockSpec((1, H), lambda i: (0, 0)),
            pl.BlockSpec((H, C), lambda i: (0, 0)),
            pl.BlockSpec((1, C), lambda i: (0, 0)),
        ],
        out_specs=pl.BlockSpec((1, C), lambda i: (0, 0)),
        out_shape=jax.ShapeDtypeStruct((1, C), jnp.float32),
        scratch_shapes=[pltpu.VMEM((1, H), jnp.float32)],
    )(h2, agg2, deg, b2r, Wfc, bfcr)

    return out


# speedup vs baseline: 9.6268x; 9.6268x over previous
import functools

import jax
import jax.numpy as jnp
from jax import lax
from jax.experimental import pallas as pl
from jax.experimental.pallas import tpu as pltpu
from jax.experimental.pallas import tpu_sc as plsc

N = 10000
E = 160000
D = 256
H = 64
C = 10

NC = 2
NS = 16
N_PAD = 10240
ROWS_PER_TILE = N_PAD // NS
CH = 128
EPT = 5120
NCHUNK = EPT // CH
EPC = EPT * NS
E_PAD = EPC * NC
DEG_W = 16

RB = 400
GRID = N // RB

_mesh = plsc.VectorSubcoreMesh(core_axis_name="c", subcore_axis_name="s")
_sc_params = pltpu.CompilerParams(use_tc_tiling_on_sc=False)


def _zero_fill(ref, ncols):
    def body(r, _):
        for j in range(ncols // 16):
            ref[r, pl.ds(j * 16, 16)] = jnp.zeros((16,), jnp.float32)
        return 0
    lax.fori_loop(0, CH, body, 0)


@functools.partial(
    pl.kernel,
    out_type=jax.ShapeDtypeStruct((NC, N_PAD, DEG_W), jnp.float32),
    mesh=_mesh,
    compiler_params=_sc_params,
    scratch_types=[
        pltpu.VMEM((CH,), jnp.int32),
        pltpu.VMEM((CH, DEG_W), jnp.float32),
        pltpu.VMEM_SHARED((N_PAD, DEG_W), jnp.float32),
    ],
)
def _sc_degree(dst_hbm, out_hbm, didx, ones_v, acc):
    c = lax.axis_index("c")
    s = lax.axis_index("s")
    _zero_fill(ones_v, DEG_W)
    for k in range(ROWS_PER_TILE // CH):
        pltpu.sync_copy(ones_v, acc.at[pl.ds(s * ROWS_PER_TILE + k * CH, CH)])
    def fill1(r, _):
        ones_v[r, pl.ds(0, 16)] = jnp.full((16,), 1.0, jnp.float32)
        return 0
    lax.fori_loop(0, CH, fill1, 0)
    plsc.subcore_barrier()

    def chunk(i, _):
        base = pl.multiple_of(c * EPC + s * EPT + i * CH, CH)
        pltpu.sync_copy(dst_hbm.at[pl.ds(base, CH)], didx)
        pltpu.sync_copy(ones_v, acc.at[didx], add=True)
        return 0
    lax.fori_loop(0, NCHUNK, chunk, 0)
    plsc.subcore_barrier()
    pltpu.sync_copy(
        acc.at[pl.ds(s * ROWS_PER_TILE, ROWS_PER_TILE)],
        out_hbm.at[c, pl.ds(s * ROWS_PER_TILE, ROWS_PER_TILE)],
    )


@functools.partial(
    pl.kernel,
    out_type=jax.ShapeDtypeStruct((NC, N_PAD, H), jnp.float32),
    mesh=_mesh,
    compiler_params=_sc_params,
    scratch_types=[
        pltpu.VMEM((CH,), jnp.int32),
        pltpu.VMEM((CH,), jnp.int32),
        pltpu.VMEM((CH, H), jnp.float32),
        pltpu.VMEM_SHARED((N_PAD, H), jnp.float32),
        pltpu.SemaphoreType.DMA,
    ],
)
def _sc_aggregate(g_hbm, src_hbm, dst_hbm, out_hbm, sidx, didx, rows, acc, sem):
    c = lax.axis_index("c")
    s = lax.axis_index("s")
    _zero_fill(rows, H)
    for k in range(ROWS_PER_TILE // CH):
        pltpu.sync_copy(rows, acc.at[pl.ds(s * ROWS_PER_TILE + k * CH, CH)])
    plsc.subcore_barrier()

    def chunk(i, _):
        base = pl.multiple_of(c * EPC + s * EPT + i * CH, CH)
        pltpu.sync_copy(src_hbm.at[pl.ds(base, CH)], sidx)
        pltpu.sync_copy(dst_hbm.at[pl.ds(base, CH)], didx)
        pltpu.async_copy(g_hbm.at[sidx], rows, sem).wait()
        pltpu.sync_copy(rows, acc.at[didx], add=True)
        return 0
    lax.fori_loop(0, NCHUNK, chunk, 0)
    plsc.subcore_barrier()
    pltpu.sync_copy(
        acc.at[pl.ds(s * ROWS_PER_TILE, ROWS_PER_TILE)],
        out_hbm.at[c, pl.ds(s * ROWS_PER_TILE, ROWS_PER_TILE)],
    )


def _dinv_block(deg_ref):
    d = deg_ref[0, :, 0:1] + deg_ref[1, :, 0:1] + 1.0
    return lax.rsqrt(d)


def _tc_mm1_body(x_ref, w_ref, deg_ref, h_ref, g_ref):
    h = jnp.dot(x_ref[...], w_ref[...], preferred_element_type=jnp.float32)
    dinv = _dinv_block(deg_ref)
    h_ref[...] = h
    g_ref[...] = h * dinv


def _tc_layer2_body(h1_ref, agg_ref, deg_ref, w2_ref, b1_ref, h2_ref, g2_ref):
    dinv = _dinv_block(deg_ref)
    agg = agg_ref[0] + agg_ref[1]
    out1 = jnp.maximum(dinv * agg + dinv * dinv * h1_ref[...] + b1_ref[...], 0.0)
    h2 = jnp.dot(out1, w2_ref[...], preferred_element_type=jnp.float32)
    h2_ref[...] = h2
    g2_ref[...] = h2 * dinv


def _tc_final_body(h2_ref, agg_ref, deg_ref, b2_ref, wfc_ref, bfc_ref,
                   out_ref, acc_ref):
    i = pl.program_id(0)
    dinv = _dinv_block(deg_ref)
    agg = agg_ref[0] + agg_ref[1]
    out2 = jnp.maximum(dinv * agg + dinv * dinv * h2_ref[...] + b2_ref[...], 0.0)
    part = jnp.sum(out2, axis=0, keepdims=True)

    @pl.when(i == 0)
    def _():
        acc_ref[...] = part

    @pl.when(i > 0)
    def _():
        acc_ref[...] = acc_ref[...] + part

    @pl.when(i == GRID - 1)
    def _():
        pooled = acc_ref[...] * (1.0 / N)
        out_ref[...] = (
            jnp.dot(pooled, wfc_ref[...], preferred_element_type=jnp.float32)
            + bfc_ref[...]
        )


def kernel(x, edge_index, W1, b1, W2, b2, Wfc, bfc):
    src = edge_index[0]
    dst = edge_index[1]
    pad = E_PAD - E
    src_p = jnp.concatenate([src, jnp.zeros((pad,), jnp.int32)])
    dst_p = jnp.concatenate([dst, jnp.full((pad,), N, jnp.int32)])
    b1r = b1.reshape(1, H)
    b2r = b2.reshape(1, H)
    bfcr = bfc.reshape(1, C)

    deg = _sc_degree(dst_p)

    deg_spec = pl.BlockSpec((NC, RB, DEG_W), lambda i: (0, i, 0))
    agg_spec = pl.BlockSpec((NC, RB, H), lambda i: (0, i, 0))
    row_spec = pl.BlockSpec((RB, H), lambda i: (i, 0))

    h1, g1 = pl.pallas_call(
        _tc_mm1_body,
        grid=(GRID,),
        in_specs=[
            pl.BlockSpec((RB, D), lambda i: (i, 0)),
            pl.BlockSpec((D, H), lambda i: (0, 0)),
            deg_spec,
        ],
        out_specs=[row_spec, row_spec],
        out_shape=[
            jax.ShapeDtypeStruct((N, H), jnp.float32),
            jax.ShapeDtypeStruct((N, H), jnp.float32),
        ],
    )(x, W1, deg)

    agg1 = _sc_aggregate(g1, src_p, dst_p)

    h2, g2 = pl.pallas_call(
        _tc_layer2_body,
        grid=(GRID,),
        in_specs=[
            row_spec,
            agg_spec,
            deg_spec,
            pl.BlockSpec((H, H), lambda i: (0, 0)),
            pl.BlockSpec((1, H), lambda i: (0, 0)),
        ],
        out_specs=[row_spec, row_spec],
        out_shape=[
            jax.ShapeDtypeStruct((N, H), jnp.float32),
            jax.ShapeDtypeStruct((N, H), jnp.float32),
        ],
    )(h1, agg1, deg, W2, b1r)

    agg2 = _sc_aggregate(g2, src_p, dst_p)

    out = pl.pallas_call(
        _tc_final_body,
        grid=(GRID,),
        in_specs=[
            row_spec,
            agg_spec,
            deg_spec,
            pl.BlockSpec((1, H), lambda i: (0, 0)),
            pl.BlockSpec((H, C), lambda i: (0, 0)),
            pl.BlockSpec((1, C), lambda i: (0, 0)),
        ],
        out_specs=pl.BlockSpec((1, C), lambda i: (0, 0)),
        out_shape=jax.ShapeDtypeStruct((1, C), jnp.float32),
        scratch_shapes=[pltpu.VMEM((1, H), jnp.float32)],
    )(h2, agg2, deg, b2r, Wfc, bfcr)

    return out


# speedup vs baseline: 12.8788x; 1.3378x over previous
import functools

import jax
import jax.numpy as jnp
from jax import lax
from jax.experimental import pallas as pl
from jax.experimental.pallas import tpu as pltpu
from jax.experimental.pallas import tpu_sc as plsc

N = 10000
E = 160000
D = 256
H = 64
C = 10

NC = 2
NS = 16
N_PAD = 10240
ROWS_PER_TILE = N_PAD // NS
CH = 128
EPT = 5120
NCHUNK = EPT // CH
EPC = EPT * NS
E_PAD = EPC * NC
DEG_W = 16

RB = 400
GRID = N // RB

_mesh = plsc.VectorSubcoreMesh(core_axis_name="c", subcore_axis_name="s")
_sc_params = pltpu.CompilerParams(use_tc_tiling_on_sc=False)


def _zero_fill(ref, ncols):
    def body(r, _):
        for j in range(ncols // 16):
            ref[r, pl.ds(j * 16, 16)] = jnp.zeros((16,), jnp.float32)
        return 0
    lax.fori_loop(0, CH, body, 0)


@functools.partial(
    pl.kernel,
    out_type=jax.ShapeDtypeStruct((NC, N_PAD, DEG_W), jnp.float32),
    mesh=_mesh,
    compiler_params=_sc_params,
    scratch_types=[
        pltpu.VMEM((NCHUNK, CH), jnp.int32),
        pltpu.VMEM((CH, DEG_W), jnp.float32),
        pltpu.VMEM_SHARED((N_PAD, DEG_W), jnp.float32),
    ],
)
def _sc_degree(dst2_hbm, out_hbm, didx_all, ones_v, acc):
    c = lax.axis_index("c")
    s = lax.axis_index("s")
    cbase = pl.multiple_of((c * EPC + s * EPT) // CH, 8)
    pltpu.sync_copy(dst2_hbm.at[pl.ds(cbase, NCHUNK)], didx_all)
    _zero_fill(ones_v, DEG_W)
    for k in range(ROWS_PER_TILE // CH):
        pltpu.sync_copy(ones_v, acc.at[pl.ds(s * ROWS_PER_TILE + k * CH, CH)])
    def fill1(r, _):
        ones_v[r, pl.ds(0, 16)] = jnp.full((16,), 1.0, jnp.float32)
        return 0
    lax.fori_loop(0, CH, fill1, 0)
    plsc.subcore_barrier()

    def chunk(i, _):
        pltpu.sync_copy(ones_v, acc.at[didx_all.at[i]], add=True)
        return 0
    lax.fori_loop(0, NCHUNK, chunk, 0)
    plsc.subcore_barrier()
    pltpu.sync_copy(
        acc.at[pl.ds(s * ROWS_PER_TILE, ROWS_PER_TILE)],
        out_hbm.at[c, pl.ds(s * ROWS_PER_TILE, ROWS_PER_TILE)],
    )


@functools.partial(
    pl.kernel,
    out_type=jax.ShapeDtypeStruct((NC, N_PAD, H), jnp.float32),
    mesh=_mesh,
    compiler_params=_sc_params,
    scratch_types=[
        pltpu.VMEM((EPT,), jnp.int32),
        pltpu.VMEM((NCHUNK, CH), jnp.int32),
        pltpu.VMEM((CH, H), jnp.float32),
        pltpu.VMEM((CH, H), jnp.float32),
        pltpu.VMEM_SHARED((N_PAD, H), jnp.float32),
        pltpu.SemaphoreType.DMA,
        pltpu.SemaphoreType.DMA,
    ],
)
def _sc_aggregate(g_hbm, src_hbm, dst2_hbm, out_hbm,
                  sidx_all, didx_all, rows0, rows1, acc, sem0, sem1):
    c = lax.axis_index("c")
    s = lax.axis_index("s")
    ebase = pl.multiple_of(c * EPC + s * EPT, CH)
    pltpu.sync_copy(src_hbm.at[pl.ds(ebase, EPT)], sidx_all)
    cbase = pl.multiple_of((c * EPC + s * EPT) // CH, 8)
    pltpu.sync_copy(dst2_hbm.at[pl.ds(cbase, NCHUNK)], didx_all)
    _zero_fill(rows0, H)
    for k in range(ROWS_PER_TILE // CH):
        pltpu.sync_copy(rows0, acc.at[pl.ds(s * ROWS_PER_TILE + k * CH, CH)])
    plsc.subcore_barrier()

    def gather(i, buf, sem):
        return pltpu.async_copy(
            g_hbm.at[sidx_all.at[pl.ds(i * CH, CH)]], buf, sem)

    def gather_wait(i, buf, sem):
        pltpu.make_async_copy(
            g_hbm.at[sidx_all.at[pl.ds(i * CH, CH)]], buf, sem).wait()

    gather(0, rows0, sem0)

    def pair(j, _):
        i = j * 2
        gather(i + 1, rows1, sem1)
        gather_wait(i, rows0, sem0)
        pltpu.sync_copy(rows0, acc.at[didx_all.at[i]], add=True)

        @pl.when(i + 2 < NCHUNK)
        def _():
            gather(i + 2, rows0, sem0)

        gather_wait(i + 1, rows1, sem1)
        pltpu.sync_copy(rows1, acc.at[didx_all.at[i + 1]], add=True)
        return 0
    lax.fori_loop(0, NCHUNK // 2, pair, 0)
    plsc.subcore_barrier()
    pltpu.sync_copy(
        acc.at[pl.ds(s * ROWS_PER_TILE, ROWS_PER_TILE)],
        out_hbm.at[c, pl.ds(s * ROWS_PER_TILE, ROWS_PER_TILE)],
    )


def _dinv_block(deg_ref):
    d = deg_ref[0, :, 0:1] + deg_ref[1, :, 0:1] + 1.0
    return lax.rsqrt(d)


def _tc_mm1_body(x_ref, w_ref, deg_ref, h_ref, g_ref):
    h = jnp.dot(x_ref[...], w_ref[...], preferred_element_type=jnp.float32)
    dinv = _dinv_block(deg_ref)
    h_ref[...] = h
    g_ref[...] = h * dinv


def _tc_layer2_body(h1_ref, agg_ref, deg_ref, w2_ref, b1_ref, h2_ref, g2_ref):
    dinv = _dinv_block(deg_ref)
    agg = agg_ref[0] + agg_ref[1]
    out1 = jnp.maximum(dinv * agg + dinv * dinv * h1_ref[...] + b1_ref[...], 0.0)
    h2 = jnp.dot(out1, w2_ref[...], preferred_element_type=jnp.float32)
    h2_ref[...] = h2
    g2_ref[...] = h2 * dinv


def _tc_final_body(h2_ref, agg_ref, deg_ref, b2_ref, wfc_ref, bfc_ref,
                   out_ref, acc_ref):
    i = pl.program_id(0)
    dinv = _dinv_block(deg_ref)
    agg = agg_ref[0] + agg_ref[1]
    out2 = jnp.maximum(dinv * agg + dinv * dinv * h2_ref[...] + b2_ref[...], 0.0)
    part = jnp.sum(out2, axis=0, keepdims=True)

    @pl.when(i == 0)
    def _():
        acc_ref[...] = part

    @pl.when(i > 0)
    def _():
        acc_ref[...] = acc_ref[...] + part

    @pl.when(i == GRID - 1)
    def _():
        pooled = acc_ref[...] * (1.0 / N)
        out_ref[...] = (
            jnp.dot(pooled, wfc_ref[...], preferred_element_type=jnp.float32)
            + bfc_ref[...]
        )


def kernel(x, edge_index, W1, b1, W2, b2, Wfc, bfc):
    src = edge_index[0]
    dst = edge_index[1]
    pad = E_PAD - E
    src_p = jnp.concatenate([src, jnp.zeros((pad,), jnp.int32)])
    dst_p = jnp.concatenate([dst, jnp.full((pad,), N, jnp.int32)])
    dst2 = dst_p.reshape(E_PAD // CH, CH)
    b1r = b1.reshape(1, H)
    b2r = b2.reshape(1, H)
    bfcr = bfc.reshape(1, C)

    deg = _sc_degree(dst2)

    deg_spec = pl.BlockSpec((NC, RB, DEG_W), lambda i: (0, i, 0))
    agg_spec = pl.BlockSpec((NC, RB, H), lambda i: (0, i, 0))
    row_spec = pl.BlockSpec((RB, H), lambda i: (i, 0))

    h1, g1 = pl.pallas_call(
        _tc_mm1_body,
        grid=(GRID,),
        in_specs=[
            pl.BlockSpec((RB, D), lambda i: (i, 0)),
            pl.BlockSpec((D, H), lambda i: (0, 0)),
            deg_spec,
        ],
        out_specs=[row_spec, row_spec],
        out_shape=[
            jax.ShapeDtypeStruct((N, H), jnp.float32),
            jax.ShapeDtypeStruct((N, H), jnp.float32),
        ],
    )(x, W1, deg)

    agg1 = _sc_aggregate(g1, src_p, dst2)

    h2, g2 = pl.pallas_call(
        _tc_layer2_body,
        grid=(GRID,),
        in_specs=[
            row_spec,
            agg_spec,
            deg_spec,
            pl.BlockSpec((H, H), lambda i: (0, 0)),
            pl.BlockSpec((1, H), lambda i: (0, 0)),
        ],
        out_specs=[row_spec, row_spec],
        out_shape=[
            jax.ShapeDtypeStruct((N, H), jnp.float32),
            jax.ShapeDtypeStruct((N, H), jnp.float32),
        ],
    )(h1, agg1, deg, W2, b1r)

    agg2 = _sc_aggregate(g2, src_p, dst2)

    out = pl.pallas_call(
        _tc_final_body,
        grid=(GRID,),
        in_specs=[
            row_spec,
            agg_spec,
            deg_spec,
            pl.BlockSpec((1, H), lambda i: (0, 0)),
            pl.BlockSpec((H, C), lambda i: (0, 0)),
            pl.BlockSpec((1, C), lambda i: (0, 0)),
        ],
        out_specs=pl.BlockSpec((1, C), lambda i: (0, 0)),
        out_shape=jax.ShapeDtypeStruct((1, C), jnp.float32),
        scratch_shapes=[pltpu.VMEM((1, H), jnp.float32)],
    )(h2, agg2, deg, b2r, Wfc, bfcr)

    return out


# speedup vs baseline: 22.8209x; 1.7720x over previous
import functools

import jax
import jax.numpy as jnp
from jax import lax
from jax.experimental import pallas as pl
from jax.experimental.pallas import tpu as pltpu
from jax.experimental.pallas import tpu_sc as plsc

N = 10000
E = 160000
D = 256
H = 64
C = 10

NC = 2
NS = 16
N_PAD = 10240
ROWS_PER_TILE = N_PAD // NS
CH = 128
EPT = 5120
NCHUNK = EPT // CH
EPC = EPT * NS
E_PAD = EPC * NC
DEG_W = 16

RB = 400
GRID = N // RB

_mesh = plsc.VectorSubcoreMesh(core_axis_name="c", subcore_axis_name="s")
_sc_params = pltpu.CompilerParams(use_tc_tiling_on_sc=False)


def _zero_fill(ref, ncols):
    def body(r, _):
        for j in range(ncols // 16):
            ref[r, pl.ds(j * 16, 16)] = jnp.zeros((16,), jnp.float32)
        return 0
    lax.fori_loop(0, CH, body, 0)


@functools.partial(
    pl.kernel,
    out_type=jax.ShapeDtypeStruct((NC, N_PAD, DEG_W), jnp.float32),
    mesh=_mesh,
    compiler_params=_sc_params,
    scratch_types=[
        pltpu.VMEM((NCHUNK, CH), jnp.int32),
        pltpu.VMEM((CH, DEG_W), jnp.float32),
        pltpu.VMEM_SHARED((N_PAD, DEG_W), jnp.float32),
    ],
)
def _sc_degree(dst2_hbm, out_hbm, didx_all, ones_v, acc):
    c = lax.axis_index("c")
    s = lax.axis_index("s")
    cbase = pl.multiple_of((c * EPC + s * EPT) // CH, 8)
    pltpu.sync_copy(dst2_hbm.at[pl.ds(cbase, NCHUNK)], didx_all)
    _zero_fill(ones_v, DEG_W)
    for k in range(ROWS_PER_TILE // CH):
        pltpu.sync_copy(ones_v, acc.at[pl.ds(s * ROWS_PER_TILE + k * CH, CH)])
    def fill1(r, _):
        ones_v[r, pl.ds(0, 16)] = jnp.full((16,), 1.0, jnp.float32)
        return 0
    lax.fori_loop(0, CH, fill1, 0)
    plsc.subcore_barrier()

    def chunk(i, _):
        pltpu.sync_copy(ones_v, acc.at[didx_all.at[i]], add=True)
        return 0
    lax.fori_loop(0, NCHUNK, chunk, 0)
    plsc.subcore_barrier()
    pltpu.sync_copy(
        acc.at[pl.ds(s * ROWS_PER_TILE, ROWS_PER_TILE)],
        out_hbm.at[c, pl.ds(s * ROWS_PER_TILE, ROWS_PER_TILE)],
    )


@functools.partial(
    pl.kernel,
    out_type=jax.ShapeDtypeStruct((NC, N_PAD, H), jnp.float32),
    mesh=_mesh,
    compiler_params=_sc_params,
    scratch_types=[
        pltpu.VMEM((EPT,), jnp.int32),
        pltpu.VMEM((NCHUNK, CH), jnp.int32),
        pltpu.VMEM((CH, H), jnp.float32),
        pltpu.VMEM((CH, H), jnp.float32),
        pltpu.VMEM_SHARED((N_PAD, H), jnp.float32),
        pltpu.SemaphoreType.DMA,
        pltpu.SemaphoreType.DMA,
    ],
)
def _sc_aggregate(g_hbm, src_hbm, dst2_hbm, out_hbm,
                  sidx_all, didx_all, rows0, rows1, acc, sem0, sem1):
    c = lax.axis_index("c")
    s = lax.axis_index("s")
    ebase = pl.multiple_of(c * EPC + s * EPT, CH)
    pltpu.sync_copy(src_hbm.at[pl.ds(ebase, EPT)], sidx_all)
    cbase = pl.multiple_of((c * EPC + s * EPT) // CH, 8)
    pltpu.sync_copy(dst2_hbm.at[pl.ds(cbase, NCHUNK)], didx_all)
    _zero_fill(rows0, H)
    for k in range(ROWS_PER_TILE // CH):
        pltpu.sync_copy(rows0, acc.at[pl.ds(s * ROWS_PER_TILE + k * CH, CH)])
    plsc.subcore_barrier()

    def gather(i, buf, sem):
        return pltpu.async_copy(
            g_hbm.at[sidx_all.at[pl.ds(i * CH, CH)]], buf, sem)

    def gather_wait(i, buf, sem):
        pltpu.make_async_copy(
            g_hbm.at[sidx_all.at[pl.ds(i * CH, CH)]], buf, sem).wait()

    gather(0, rows0, sem0)

    def pair(j, _):
        i = j * 2
        gather(i + 1, rows1, sem1)
        gather_wait(i, rows0, sem0)
        pltpu.sync_copy(rows0, acc.at[didx_all.at[i]], add=True)

        @pl.when(i + 2 < NCHUNK)
        def _():
            gather(i + 2, rows0, sem0)

        gather_wait(i + 1, rows1, sem1)
        pltpu.sync_copy(rows1, acc.at[didx_all.at[i + 1]], add=True)
        return 0
    lax.fori_loop(0, NCHUNK // 2, pair, 0)
    plsc.subcore_barrier()
    pltpu.sync_copy(
        acc.at[pl.ds(s * ROWS_PER_TILE, ROWS_PER_TILE)],
        out_hbm.at[c, pl.ds(s * ROWS_PER_TILE, ROWS_PER_TILE)],
    )


def _dinv_block(deg_ref):
    d = deg_ref[0, :, 0:1] + deg_ref[1, :, 0:1] + 1.0
    return lax.rsqrt(d)


def _tc_mm1_body(x_ref, w_ref, deg_ref, h_ref, g_ref):
    h = jnp.dot(x_ref[...], w_ref[...], preferred_element_type=jnp.float32)
    dinv = _dinv_block(deg_ref)
    h_ref[...] = h
    g_ref[...] = h * dinv


def _tc_layer2_body(h1_ref, agg_ref, deg_ref, w2_ref, b1_ref, h2_ref, g2_ref):
    dinv = _dinv_block(deg_ref)
    agg = agg_ref[0] + agg_ref[1]
    out1 = jnp.maximum(dinv * agg + dinv * dinv * h1_ref[...] + b1_ref[...], 0.0)
    h2 = jnp.dot(out1, w2_ref[...], preferred_element_type=jnp.float32)
    h2_ref[...] = h2
    g2_ref[...] = h2 * dinv


def _tc_final_body(h2_ref, agg_ref, deg_ref, b2_ref, wfc_ref, bfc_ref,
                   out_ref, acc_ref):
    i = pl.program_id(0)
    dinv = _dinv_block(deg_ref)
    agg = agg_ref[0] + agg_ref[1]
    out2 = jnp.maximum(dinv * agg + dinv * dinv * h2_ref[...] + b2_ref[...], 0.0)
    part = jnp.sum(out2, axis=0, keepdims=True)

    @pl.when(i == 0)
    def _():
        acc_ref[...] = part

    @pl.when(i > 0)
    def _():
        acc_ref[...] = acc_ref[...] + part

    @pl.when(i == GRID - 1)
    def _():
        pooled = acc_ref[...] * (1.0 / N)
        out_ref[...] = (
            jnp.dot(pooled, wfc_ref[...], preferred_element_type=jnp.float32)
            + bfc_ref[...]
        )


def kernel(x, edge_index, W1, b1, W2, b2, Wfc, bfc):
    src = edge_index[0]
    dst = edge_index[1]
    pad = E_PAD - E
    fill = jnp.arange(pad, dtype=jnp.int32)
    src_p = jnp.concatenate([src, fill % N])
    dst_p = jnp.concatenate([dst, N + (fill % (N_PAD - N))])
    dst2 = dst_p.reshape(E_PAD // CH, CH)
    b1r = b1.reshape(1, H)
    b2r = b2.reshape(1, H)
    bfcr = bfc.reshape(1, C)

    deg = _sc_degree(dst2)

    deg_spec = pl.BlockSpec((NC, RB, DEG_W), lambda i: (0, i, 0))
    agg_spec = pl.BlockSpec((NC, RB, H), lambda i: (0, i, 0))
    row_spec = pl.BlockSpec((RB, H), lambda i: (i, 0))

    h1, g1 = pl.pallas_call(
        _tc_mm1_body,
        grid=(GRID,),
        in_specs=[
            pl.BlockSpec((RB, D), lambda i: (i, 0)),
            pl.BlockSpec((D, H), lambda i: (0, 0)),
            deg_spec,
        ],
        out_specs=[row_spec, row_spec],
        out_shape=[
            jax.ShapeDtypeStruct((N, H), jnp.float32),
            jax.ShapeDtypeStruct((N, H), jnp.float32),
        ],
    )(x, W1, deg)

    agg1 = _sc_aggregate(g1, src_p, dst2)

    h2, g2 = pl.pallas_call(
        _tc_layer2_body,
        grid=(GRID,),
        in_specs=[
            row_spec,
            agg_spec,
            deg_spec,
            pl.BlockSpec((H, H), lambda i: (0, 0)),
            pl.BlockSpec((1, H), lambda i: (0, 0)),
        ],
        out_specs=[row_spec, row_spec],
        out_shape=[
            jax.ShapeDtypeStruct((N, H), jnp.float32),
            jax.ShapeDtypeStruct((N, H), jnp.float32),
        ],
    )(h1, agg1, deg, W2, b1r)

    agg2 = _sc_aggregate(g2, src_p, dst2)

    out = pl.pallas_call(
        _tc_final_body,
        grid=(GRID,),
        in_specs=[
            row_spec,
            agg_spec,
            deg_spec,
            pl.BlockSpec((1, H), lambda i: (0, 0)),
            pl.BlockSpec((H, C), lambda i: (0, 0)),
            pl.BlockSpec((1, C), lambda i: (0, 0)),
        ],
        out_specs=pl.BlockSpec((1, C), lambda i: (0, 0)),
        out_shape=jax.ShapeDtypeStruct((1, C), jnp.float32),
        scratch_shapes=[pltpu.VMEM((1, H), jnp.float32)],
    )(h2, agg2, deg, b2r, Wfc, bfcr)

    return out


# speedup vs baseline: 26.6165x; 1.1663x over previous
import functools

import jax
import jax.numpy as jnp
from jax import lax
from jax.experimental import pallas as pl
from jax.experimental.pallas import tpu as pltpu
from jax.experimental.pallas import tpu_sc as plsc

N = 10000
E = 160000
D = 256
H = 64
C = 10

NC = 2
NS = 16
N_PAD = 10240
ROWS_PER_TILE = N_PAD // NS
CH = 128
EPT = 5120
NCHUNK = EPT // CH
NCHUNK_LAST = (E - 31 * EPT) // CH
NROWS2 = E // CH
DEG_W = 16

RB = 2000
GRID = N // RB

_mesh = plsc.VectorSubcoreMesh(core_axis_name="c", subcore_axis_name="s")
_sc_params = pltpu.CompilerParams(use_tc_tiling_on_sc=False)


def _zero_fill(ref, ncols):
    def body(r, _):
        for j in range(ncols // 16):
            ref[r, pl.ds(j * 16, 16)] = jnp.zeros((16,), jnp.float32)
        return 0
    lax.fori_loop(0, CH, body, 0)


@functools.partial(
    pl.kernel,
    out_type=jax.ShapeDtypeStruct((NC, N_PAD, DEG_W), jnp.float32),
    mesh=_mesh,
    compiler_params=_sc_params,
    scratch_types=[
        pltpu.VMEM((NCHUNK, CH), jnp.int32),
        pltpu.VMEM((CH, DEG_W), jnp.float32),
        pltpu.VMEM_SHARED((N_PAD, DEG_W), jnp.float32),
    ],
)
def _sc_degree(dst2_hbm, out_hbm, didx_all, ones_v, acc):
    c = lax.axis_index("c")
    s = lax.axis_index("s")
    w = c * NS + s
    nch = jnp.where(w == NC * NS - 1, NCHUNK_LAST, NCHUNK)
    cbase = pl.multiple_of(w * NCHUNK, 8)

    @pl.when(w < NC * NS - 1)
    def _():
        pltpu.sync_copy(dst2_hbm.at[pl.ds(cbase, NCHUNK)], didx_all)

    @pl.when(w == NC * NS - 1)
    def _():
        pltpu.sync_copy(dst2_hbm.at[pl.ds(cbase, NCHUNK_LAST)],
                        didx_all.at[pl.ds(0, NCHUNK_LAST)])
    _zero_fill(ones_v, DEG_W)
    for k in range(ROWS_PER_TILE // CH):
        pltpu.sync_copy(ones_v, acc.at[pl.ds(s * ROWS_PER_TILE + k * CH, CH)])
    def fill1(r, _):
        ones_v[r, pl.ds(0, 16)] = jnp.full((16,), 1.0, jnp.float32)
        return 0
    lax.fori_loop(0, CH, fill1, 0)
    plsc.subcore_barrier()

    def chunk(i, _):
        pltpu.sync_copy(ones_v, acc.at[didx_all.at[i]], add=True)
        return 0
    lax.fori_loop(0, nch, chunk, 0)
    plsc.subcore_barrier()
    pltpu.sync_copy(
        acc.at[pl.ds(s * ROWS_PER_TILE, ROWS_PER_TILE)],
        out_hbm.at[c, pl.ds(s * ROWS_PER_TILE, ROWS_PER_TILE)],
    )


@functools.partial(
    pl.kernel,
    out_type=jax.ShapeDtypeStruct((NC, N_PAD, H), jnp.float32),
    mesh=_mesh,
    compiler_params=_sc_params,
    scratch_types=[
        pltpu.VMEM((EPT,), jnp.int32),
        pltpu.VMEM((NCHUNK, CH), jnp.int32),
        pltpu.VMEM((CH, H), jnp.float32),
        pltpu.VMEM((CH, H), jnp.float32),
        pltpu.VMEM_SHARED((N_PAD, H), jnp.float32),
        pltpu.SemaphoreType.DMA,
        pltpu.SemaphoreType.DMA,
    ],
)
def _sc_aggregate(g_hbm, src_hbm, dst2_hbm, out_hbm,
                  sidx_all, didx_all, rows0, rows1, acc, sem0, sem1):
    c = lax.axis_index("c")
    s = lax.axis_index("s")
    w = c * NS + s
    nch = jnp.where(w == NC * NS - 1, NCHUNK_LAST, NCHUNK)
    ebase = pl.multiple_of(w * EPT, CH)
    cbase = pl.multiple_of(w * NCHUNK, 8)

    @pl.when(w < NC * NS - 1)
    def _():
        pltpu.sync_copy(src_hbm.at[pl.ds(ebase, EPT)], sidx_all)
        pltpu.sync_copy(dst2_hbm.at[pl.ds(cbase, NCHUNK)], didx_all)

    @pl.when(w == NC * NS - 1)
    def _():
        pltpu.sync_copy(src_hbm.at[pl.ds(ebase, NCHUNK_LAST * CH)],
                        sidx_all.at[pl.ds(0, NCHUNK_LAST * CH)])
        pltpu.sync_copy(dst2_hbm.at[pl.ds(cbase, NCHUNK_LAST)],
                        didx_all.at[pl.ds(0, NCHUNK_LAST)])
    _zero_fill(rows0, H)
    for k in range(ROWS_PER_TILE // CH):
        pltpu.sync_copy(rows0, acc.at[pl.ds(s * ROWS_PER_TILE + k * CH, CH)])
    plsc.subcore_barrier()

    def gather(i, buf, sem):
        return pltpu.async_copy(
            g_hbm.at[sidx_all.at[pl.ds(i * CH, CH)]], buf, sem)

    def gather_wait(i, buf, sem):
        pltpu.make_async_copy(
            g_hbm.at[sidx_all.at[pl.ds(i * CH, CH)]], buf, sem).wait()

    gather(0, rows0, sem0)

    def pair(j, _):
        i = j * 2
        gather(i + 1, rows1, sem1)
        gather_wait(i, rows0, sem0)
        pltpu.sync_copy(rows0, acc.at[didx_all.at[i]], add=True)

        @pl.when(i + 2 < nch)
        def _():
            gather(i + 2, rows0, sem0)

        gather_wait(i + 1, rows1, sem1)
        pltpu.sync_copy(rows1, acc.at[didx_all.at[i + 1]], add=True)
        return 0
    lax.fori_loop(0, nch // 2, pair, 0)
    plsc.subcore_barrier()
    pltpu.sync_copy(
        acc.at[pl.ds(s * ROWS_PER_TILE, ROWS_PER_TILE)],
        out_hbm.at[c, pl.ds(s * ROWS_PER_TILE, ROWS_PER_TILE)],
    )


def _dinv_block(deg_ref):
    d = deg_ref[0, :, 0:1] + deg_ref[1, :, 0:1] + 1.0
    return lax.rsqrt(d)


def _tc_mm1_body(x_ref, w_ref, deg_ref, h_ref, g_ref):
    h = jnp.dot(x_ref[...], w_ref[...], preferred_element_type=jnp.float32)
    dinv = _dinv_block(deg_ref)
    h_ref[...] = h
    g_ref[...] = h * dinv


def _tc_layer2_body(h1_ref, agg_ref, deg_ref, w2_ref, b1_ref, h2_ref, g2_ref):
    dinv = _dinv_block(deg_ref)
    agg = agg_ref[0] + agg_ref[1]
    out1 = jnp.maximum(dinv * agg + dinv * dinv * h1_ref[...] + b1_ref[...], 0.0)
    h2 = jnp.dot(out1, w2_ref[...], preferred_element_type=jnp.float32)
    h2_ref[...] = h2
    g2_ref[...] = h2 * dinv


def _tc_final_body(h2_ref, agg_ref, deg_ref, b2_ref, wfc_ref, bfc_ref,
                   out_ref, acc_ref):
    i = pl.program_id(0)
    dinv = _dinv_block(deg_ref)
    agg = agg_ref[0] + agg_ref[1]
    out2 = jnp.maximum(dinv * agg + dinv * dinv * h2_ref[...] + b2_ref[...], 0.0)
    part = jnp.sum(out2, axis=0, keepdims=True)

    @pl.when(i == 0)
    def _():
        acc_ref[...] = part

    @pl.when(i > 0)
    def _():
        acc_ref[...] = acc_ref[...] + part

    @pl.when(i == GRID - 1)
    def _():
        pooled = acc_ref[...] * (1.0 / N)
        out_ref[...] = (
            jnp.dot(pooled, wfc_ref[...], preferred_element_type=jnp.float32)
            + bfc_ref[...]
        )


def kernel(x, edge_index, W1, b1, W2, b2, Wfc, bfc):
    src_p = edge_index[0]
    dst2 = edge_index[1].reshape(NROWS2, CH)
    b1r = b1.reshape(1, H)
    b2r = b2.reshape(1, H)
    bfcr = bfc.reshape(1, C)

    deg = _sc_degree(dst2)

    deg_spec = pl.BlockSpec((NC, RB, DEG_W), lambda i: (0, i, 0))
    agg_spec = pl.BlockSpec((NC, RB, H), lambda i: (0, i, 0))
    row_spec = pl.BlockSpec((RB, H), lambda i: (i, 0))

    h1, g1 = pl.pallas_call(
        _tc_mm1_body,
        grid=(GRID,),
        in_specs=[
            pl.BlockSpec((RB, D), lambda i: (i, 0)),
            pl.BlockSpec((D, H), lambda i: (0, 0)),
            deg_spec,
        ],
        out_specs=[row_spec, row_spec],
        out_shape=[
            jax.ShapeDtypeStruct((N, H), jnp.float32),
            jax.ShapeDtypeStruct((N, H), jnp.float32),
        ],
    )(x, W1, deg)

    agg1 = _sc_aggregate(g1, src_p, dst2)

    h2, g2 = pl.pallas_call(
        _tc_layer2_body,
        grid=(GRID,),
        in_specs=[
            row_spec,
            agg_spec,
            deg_spec,
            pl.BlockSpec((H, H), lambda i: (0, 0)),
            pl.BlockSpec((1, H), lambda i: (0, 0)),
        ],
        out_specs=[row_spec, row_spec],
        out_shape=[
            jax.ShapeDtypeStruct((N, H), jnp.float32),
            jax.ShapeDtypeStruct((N, H), jnp.float32),
        ],
    )(h1, agg1, deg, W2, b1r)

    agg2 = _sc_aggregate(g2, src_p, dst2)

    out = pl.pallas_call(
        _tc_final_body,
        grid=(GRID,),
        in_specs=[
            row_spec,
            agg_spec,
            deg_spec,
            pl.BlockSpec((1, H), lambda i: (0, 0)),
            pl.BlockSpec((H, C), lambda i: (0, 0)),
            pl.BlockSpec((1, C), lambda i: (0, 0)),
        ],
        out_specs=pl.BlockSpec((1, C), lambda i: (0, 0)),
        out_shape=jax.ShapeDtypeStruct((1, C), jnp.float32),
        scratch_shapes=[pltpu.VMEM((1, H), jnp.float32)],
    )(h2, agg2, deg, b2r, Wfc, bfcr)

    return out


# speedup vs baseline: 28.3174x; 1.0639x over previous
import functools

import jax
import jax.numpy as jnp
from jax import lax
from jax.experimental import pallas as pl
from jax.experimental.pallas import tpu as pltpu
from jax.experimental.pallas import tpu_sc as plsc

N = 10000
E = 160000
D = 256
H = 64
C = 10

NC = 2
NS = 16
N_PAD = 10240
ROWS_PER_TILE = N_PAD // NS
CH = 128
EPT = 5120
NCHUNK = EPT // CH
NCHUNK_LAST = (E - 31 * EPT) // CH
NROWS2 = E // CH
DEG_W = 16

RB = 2000
GRID = N // RB

_mesh = plsc.VectorSubcoreMesh(core_axis_name="c", subcore_axis_name="s")
_sc_params = pltpu.CompilerParams(use_tc_tiling_on_sc=False)


def _zero_fill(ref, ncols):
    def body(r, _):
        for j in range(ncols // 16):
            ref[r, pl.ds(j * 16, 16)] = jnp.zeros((16,), jnp.float32)
        return 0
    lax.fori_loop(0, CH, body, 0)


@functools.partial(
    pl.kernel,
    out_type=jax.ShapeDtypeStruct((NC, N_PAD, DEG_W), jnp.float32),
    mesh=_mesh,
    compiler_params=_sc_params,
    scratch_types=[
        pltpu.VMEM((NCHUNK, CH), jnp.int32),
        pltpu.VMEM((CH, DEG_W), jnp.float32),
        pltpu.VMEM_SHARED((N_PAD, DEG_W), jnp.float32),
    ],
)
def _sc_degree(ei3_hbm, out_hbm, didx_all, ones_v, acc):
    c = lax.axis_index("c")
    s = lax.axis_index("s")
    w = c * NS + s
    nch = jnp.where(w == NC * NS - 1, NCHUNK_LAST, NCHUNK)
    cbase = pl.multiple_of(w * NCHUNK, 8)

    @pl.when(w < NC * NS - 1)
    def _():
        pltpu.sync_copy(ei3_hbm.at[1, pl.ds(cbase, NCHUNK)], didx_all)

    @pl.when(w == NC * NS - 1)
    def _():
        pltpu.sync_copy(ei3_hbm.at[1, pl.ds(cbase, NCHUNK_LAST)],
                        didx_all.at[pl.ds(0, NCHUNK_LAST)])
    _zero_fill(ones_v, DEG_W)
    for k in range(ROWS_PER_TILE // CH):
        pltpu.sync_copy(ones_v, acc.at[pl.ds(s * ROWS_PER_TILE + k * CH, CH)])
    def fill1(r, _):
        ones_v[r, pl.ds(0, 16)] = jnp.full((16,), 1.0, jnp.float32)
        return 0
    lax.fori_loop(0, CH, fill1, 0)
    plsc.subcore_barrier()

    def chunk(i, _):
        pltpu.sync_copy(ones_v, acc.at[didx_all.at[i]], add=True)
        return 0
    lax.fori_loop(0, nch, chunk, 0)
    plsc.subcore_barrier()
    pltpu.sync_copy(
        acc.at[pl.ds(s * ROWS_PER_TILE, ROWS_PER_TILE)],
        out_hbm.at[c, pl.ds(s * ROWS_PER_TILE, ROWS_PER_TILE)],
    )


@functools.partial(
    pl.kernel,
    out_type=jax.ShapeDtypeStruct((NC, N_PAD, H), jnp.float32),
    mesh=_mesh,
    compiler_params=_sc_params,
    scratch_types=[
        pltpu.VMEM((NCHUNK, CH), jnp.int32),
        pltpu.VMEM((NCHUNK, CH), jnp.int32),
        pltpu.VMEM((CH, H), jnp.float32),
        pltpu.VMEM((CH, H), jnp.float32),
        pltpu.VMEM_SHARED((N_PAD, H), jnp.float32),
        pltpu.SemaphoreType.DMA,
        pltpu.SemaphoreType.DMA,
    ],
)
def _sc_aggregate(g_hbm, ei3_hbm, out_hbm,
                  sidx_all, didx_all, rows0, rows1, acc, sem0, sem1):
    c = lax.axis_index("c")
    s = lax.axis_index("s")
    w = c * NS + s
    nch = jnp.where(w == NC * NS - 1, NCHUNK_LAST, NCHUNK)
    cbase = pl.multiple_of(w * NCHUNK, 8)

    @pl.when(w < NC * NS - 1)
    def _():
        pltpu.sync_copy(ei3_hbm.at[0, pl.ds(cbase, NCHUNK)], sidx_all)
        pltpu.sync_copy(ei3_hbm.at[1, pl.ds(cbase, NCHUNK)], didx_all)

    @pl.when(w == NC * NS - 1)
    def _():
        pltpu.sync_copy(ei3_hbm.at[0, pl.ds(cbase, NCHUNK_LAST)],
                        sidx_all.at[pl.ds(0, NCHUNK_LAST)])
        pltpu.sync_copy(ei3_hbm.at[1, pl.ds(cbase, NCHUNK_LAST)],
                        didx_all.at[pl.ds(0, NCHUNK_LAST)])
    _zero_fill(rows0, H)
    for k in range(ROWS_PER_TILE // CH):
        pltpu.sync_copy(rows0, acc.at[pl.ds(s * ROWS_PER_TILE + k * CH, CH)])
    plsc.subcore_barrier()

    def gather(i, buf, sem):
        return pltpu.async_copy(g_hbm.at[sidx_all.at[i]], buf, sem)

    def gather_wait(i, buf, sem):
        pltpu.make_async_copy(g_hbm.at[sidx_all.at[i]], buf, sem).wait()

    gather(0, rows0, sem0)

    def pair(j, _):
        i = j * 2
        gather(i + 1, rows1, sem1)
        gather_wait(i, rows0, sem0)
        pltpu.sync_copy(rows0, acc.at[didx_all.at[i]], add=True)

        @pl.when(i + 2 < nch)
        def _():
            gather(i + 2, rows0, sem0)

        gather_wait(i + 1, rows1, sem1)
        pltpu.sync_copy(rows1, acc.at[didx_all.at[i + 1]], add=True)
        return 0
    lax.fori_loop(0, nch // 2, pair, 0)
    plsc.subcore_barrier()
    pltpu.sync_copy(
        acc.at[pl.ds(s * ROWS_PER_TILE, ROWS_PER_TILE)],
        out_hbm.at[c, pl.ds(s * ROWS_PER_TILE, ROWS_PER_TILE)],
    )


def _dinv_block(deg_ref):
    d = deg_ref[0, :, 0:1] + deg_ref[1, :, 0:1] + 1.0
    return lax.rsqrt(d)


def _tc_mm1_body(x_ref, w_ref, h_ref):
    h_ref[...] = jnp.dot(x_ref[...], w_ref[...],
                         preferred_element_type=jnp.float32)


def _tc_scale_body(h_ref, deg_ref, g_ref):
    g_ref[...] = h_ref[...] * _dinv_block(deg_ref)


def _tc_layer2_body(h1_ref, agg_ref, deg_ref, w2_ref, b1_ref, g2_ref):
    dinv = _dinv_block(deg_ref)
    d2 = dinv * dinv
    agg = agg_ref[0] + agg_ref[1]
    u1 = jnp.maximum(d2 * agg + d2 * dinv * h1_ref[...] + dinv * b1_ref[...],
                     0.0)
    g2_ref[...] = jnp.dot(u1, w2_ref[...], preferred_element_type=jnp.float32)


def _tc_final_body(g2_ref, agg_ref, deg_ref, b2_ref, wfc_ref, bfc_ref,
                   out_ref, acc_ref):
    i = pl.program_id(0)
    dinv = _dinv_block(deg_ref)
    agg = agg_ref[0] + agg_ref[1]
    out2 = jnp.maximum(dinv * (agg + g2_ref[...]) + b2_ref[...], 0.0)
    part = jnp.sum(out2, axis=0, keepdims=True)

    @pl.when(i == 0)
    def _():
        acc_ref[...] = part

    @pl.when(i > 0)
    def _():
        acc_ref[...] = acc_ref[...] + part

    @pl.when(i == GRID - 1)
    def _():
        pooled = acc_ref[...] * (1.0 / N)
        out_ref[...] = (
            jnp.dot(pooled, wfc_ref[...], preferred_element_type=jnp.float32)
            + bfc_ref[...]
        )


def kernel(x, edge_index, W1, b1, W2, b2, Wfc, bfc):
    ei3 = edge_index.reshape(2, NROWS2, CH)
    b1r = b1.reshape(1, H)
    b2r = b2.reshape(1, H)
    bfcr = bfc.reshape(1, C)

    deg_spec = pl.BlockSpec((NC, RB, DEG_W), lambda i: (0, i, 0))
    agg_spec = pl.BlockSpec((NC, RB, H), lambda i: (0, i, 0))
    row_spec = pl.BlockSpec((RB, H), lambda i: (i, 0))
    out_nh = jax.ShapeDtypeStruct((N, H), jnp.float32)

    deg = _sc_degree(ei3)
    h1 = pl.pallas_call(
        _tc_mm1_body,
        grid=(GRID,),
        in_specs=[
            pl.BlockSpec((RB, D), lambda i: (i, 0)),
            pl.BlockSpec((D, H), lambda i: (0, 0)),
        ],
        out_specs=row_spec,
        out_shape=out_nh,
    )(x, W1)

    g1 = pl.pallas_call(
        _tc_scale_body,
        grid=(GRID,),
        in_specs=[row_spec, deg_spec],
        out_specs=row_spec,
        out_shape=out_nh,
    )(h1, deg)

    agg1 = _sc_aggregate(g1, ei3)

    g2 = pl.pallas_call(
        _tc_layer2_body,
        grid=(GRID,),
        in_specs=[
            row_spec,
            agg_spec,
            deg_spec,
            pl.BlockSpec((H, H), lambda i: (0, 0)),
            pl.BlockSpec((1, H), lambda i: (0, 0)),
        ],
        out_specs=row_spec,
        out_shape=out_nh,
    )(h1, agg1, deg, W2, b1r)

    agg2 = _sc_aggregate(g2, ei3)

    out = pl.pallas_call(
        _tc_final_body,
        grid=(GRID,),
        in_specs=[
            row_spec,
            agg_spec,
            deg_spec,
            pl.BlockSpec((1, H), lambda i: (0, 0)),
            pl.BlockSpec((H, C), lambda i: (0, 0)),
            pl.BlockSpec((1, C), lambda i: (0, 0)),
        ],
        out_specs=pl.BlockSpec((1, C), lambda i: (0, 0)),
        out_shape=jax.ShapeDtypeStruct((1, C), jnp.float32),
        scratch_shapes=[pltpu.VMEM((1, H), jnp.float32)],
    )(g2, agg2, deg, b2r, Wfc, bfcr)

    return out
